# baseline (speedup 1.0000x reference)
import functools

import jax
import jax.numpy as jnp
from jax import lax
from jax.experimental import pallas as pl
from jax.experimental.pallas import tpu as pltpu
from jax.experimental.pallas import tpu_sc as plsc

_NC = 2
_NS = 16
_NW = _NC * _NS

_OC = 4096


def _gather_cols(tabT, idxT):
    r_total, v = tabT.shape
    f_total, b_total = idxT.shape
    d = r_total // f_total
    rows_per_w = r_total // _NW
    n_chunks = b_total // _OC

    mesh = plsc.VectorSubcoreMesh(core_axis_name="c", subcore_axis_name="s")

    @functools.partial(
        pl.kernel,
        mesh=mesh,
        out_type=jax.ShapeDtypeStruct((r_total, b_total), jnp.float32),
        scratch_types=[
            pltpu.VMEM((v,), jnp.float32),
            pltpu.VMEM((b_total,), jnp.int32),
            pltpu.VMEM((_OC,), jnp.float32),
            pltpu.VMEM((_OC,), jnp.float32),
            pltpu.SemaphoreType.DMA,
            pltpu.SemaphoreType.DMA,
            pltpu.SemaphoreType.DMA,
            pltpu.SemaphoreType.DMA,
        ],
        compiler_params=pltpu.CompilerParams(needs_layout_passes=False),
    )
    def k(tab_hbm, idx_hbm, out_hbm, row_v, idx_v, out0_v, out1_v, sem0, sem1,
          sem_row0, sem_row1):
        wid = lax.axis_index("s") * _NC + lax.axis_index("c")
        out_bufs = (out0_v, out1_v)
        sems = (sem0, sem1)
        vh = v // 2

        g8 = wid // 8
        s8 = wid % 8

        def row_body(rr, f_prev):
            r = (g8 * rows_per_w + rr) * 8 + s8
            f = r // d

            row_cp = pltpu.make_async_copy(tab_hbm.at[r], row_v, sem_row0)
            row_cp.start()

            @pl.when(f != f_prev)
            def _():
                pltpu.sync_copy(idx_hbm.at[f], idx_v)

            @pl.when(rr > 0)
            def _():
                for h in range(n_chunks - 2, n_chunks):
                    pltpu.make_async_copy(
                        out_bufs[h % 2],
                        out_hbm.at[r, pl.ds(h * _OC, _OC)],
                        sems[h % 2],
                    ).wait()

            row_cp.wait()

            for h in range(n_chunks):
                buf = out_bufs[h % 2]
                sem = sems[h % 2]
                if h >= 2:
                    pltpu.make_async_copy(
                        buf, out_hbm.at[r, pl.ds((h - 2) * _OC, _OC)], sem
                    ).wait()

                @plsc.parallel_loop(0, _OC, 16, unroll=32)
                def gi(i):
                    idx16 = idx_v[pl.ds(h * _OC + i, 16)]
                    buf[pl.ds(i, 16)] = plsc.load_gather(row_v, [idx16])

                pltpu.make_async_copy(
                    buf, out_hbm.at[r, pl.ds(h * _OC, _OC)], sem
                ).start()

            return f

        lax.fori_loop(0, rows_per_w, row_body, jnp.int32(-1))
        for h in range(n_chunks - 2, n_chunks):
            pltpu.make_async_copy(
                out_bufs[h % 2],
                out_hbm.at[0, pl.ds(h * _OC, _OC)],
                sems[h % 2],
            ).wait()

    return k(tabT, idxT)


def _mm_body(xt_ref, w_ref, m_ref, o_ref):
    w = w_ref[...] * m_ref[...]
    o_ref[...] = lax.dot_general(
        w, xt_ref[...],
        dimension_numbers=(((0,), (0,)), ((), ())),
        preferred_element_type=jnp.float32,
    )


def _matmul_t(xt, w2, m2, bm):
    fd, b = xt.shape
    a = w2.shape[1]
    return pl.pallas_call(
        _mm_body,
        grid=(b // bm,),
        in_specs=[
            pl.BlockSpec((fd, bm), lambda i: (0, i)),
            pl.BlockSpec((fd, a), lambda i: (0, 0)),
            pl.BlockSpec((fd, 1), lambda i: (0, 0)),
        ],
        out_specs=pl.BlockSpec((a, bm), lambda i: (0, i)),
        out_shape=jax.ShapeDtypeStruct((a, b), jnp.float32),
    )(xt, w2, m2)


def kernel(inputs, tables, weights, sample_dims):
    f, v, d = tables.shape
    b = inputs.shape[0]
    a = weights.shape[-1]
    sample_dims = jnp.asarray(sample_dims, dtype=jnp.int32)

    tabT = jnp.transpose(tables, (0, 2, 1)).reshape(f * d, v)
    idxT = jnp.transpose(inputs, (1, 0))

    xt = _gather_cols(tabT, idxT)

    col = jnp.arange(d, dtype=jnp.int32)
    mask = (col[None, :] < sample_dims[:, None]).astype(jnp.float32)
    m2 = mask.reshape(f * d, 1)
    w2 = weights.reshape(f * d, a)
    ot = _matmul_t(xt, w2, m2, bm=2048)
    return jnp.transpose(ot, (1, 0))


# speedup vs baseline: 1.0359x; 1.0359x over previous
import functools

import jax
import jax.numpy as jnp
from jax import lax
from jax.experimental import pallas as pl
from jax.experimental.pallas import tpu as pltpu
from jax.experimental.pallas import tpu_sc as plsc

_NC = 2
_NS = 16
_NW = _NC * _NS

_OC = 4096


def _gather_cols(tabT, idxT):
    r_total, v = tabT.shape
    f_total, b_total = idxT.shape
    d = r_total // f_total
    rows_per_w = r_total // _NW
    n_chunks = b_total // _OC

    mesh = plsc.VectorSubcoreMesh(core_axis_name="c", subcore_axis_name="s")

    @functools.partial(
        pl.kernel,
        mesh=mesh,
        out_type=jax.ShapeDtypeStruct((r_total, b_total), jnp.float32),
        scratch_types=[
            pltpu.VMEM((v,), jnp.float32),
            pltpu.VMEM((b_total,), jnp.int32),
            pltpu.VMEM((_OC,), jnp.float32),
            pltpu.VMEM((_OC,), jnp.float32),
            pltpu.SemaphoreType.DMA,
            pltpu.SemaphoreType.DMA,
            pltpu.SemaphoreType.DMA,
            pltpu.SemaphoreType.DMA,
        ],
        compiler_params=pltpu.CompilerParams(needs_layout_passes=False),
    )
    def k(tab_hbm, idx_hbm, out_hbm, row_v, idx_v, out0_v, out1_v, sem0, sem1,
          sem_row0, sem_row1):
        wid = lax.axis_index("s") * _NC + lax.axis_index("c")
        out_bufs = (out0_v, out1_v)
        sems = (sem0, sem1)
        vh = v // 2

        def row_body(rr, f_prev):
            r = wid * rows_per_w + rr
            f = r // d

            row_cp = pltpu.make_async_copy(tab_hbm.at[r], row_v, sem_row0)
            row_cp.start()

            @pl.when(f != f_prev)
            def _():
                pltpu.sync_copy(idx_hbm.at[f], idx_v)

            @pl.when(rr > 0)
            def _():
                for h in range(n_chunks - 2, n_chunks):
                    pltpu.make_async_copy(
                        out_bufs[h % 2],
                        out_hbm.at[r, pl.ds(h * _OC, _OC)],
                        sems[h % 2],
                    ).wait()

            row_cp.wait()

            for h in range(n_chunks):
                buf = out_bufs[h % 2]
                sem = sems[h % 2]
                if h >= 2:
                    pltpu.make_async_copy(
                        buf, out_hbm.at[r, pl.ds((h - 2) * _OC, _OC)], sem
                    ).wait()

                @plsc.parallel_loop(0, _OC, 16, unroll=32)
                def gi(i):
                    idx16 = idx_v[pl.ds(h * _OC + i, 16)]
                    buf[pl.ds(i, 16)] = plsc.load_gather(row_v, [idx16])

                pltpu.make_async_copy(
                    buf, out_hbm.at[r, pl.ds(h * _OC, _OC)], sem
                ).start()

            return f

        lax.fori_loop(0, rows_per_w, row_body, jnp.int32(-1))
        for h in range(n_chunks - 2, n_chunks):
            pltpu.make_async_copy(
                out_bufs[h % 2],
                out_hbm.at[0, pl.ds(h * _OC, _OC)],
                sems[h % 2],
            ).wait()

    return k(tabT, idxT)


def _mm_body(xt_ref, w_ref, m_ref, o_ref):
    w = w_ref[...] * m_ref[...]
    o_ref[...] = lax.dot_general(
        w, xt_ref[...],
        dimension_numbers=(((0,), (0,)), ((), ())),
        preferred_element_type=jnp.float32,
    )


def _matmul_t(xt, w2, m2, bm):
    fd, b = xt.shape
    a = w2.shape[1]
    return pl.pallas_call(
        _mm_body,
        grid=(b // bm,),
        in_specs=[
            pl.BlockSpec((fd, bm), lambda i: (0, i)),
            pl.BlockSpec((fd, a), lambda i: (0, 0)),
            pl.BlockSpec((fd, 1), lambda i: (0, 0)),
        ],
        out_specs=pl.BlockSpec((a, bm), lambda i: (0, i)),
        out_shape=jax.ShapeDtypeStruct((a, b), jnp.float32),
    )(xt, w2, m2)


def kernel(inputs, tables, weights, sample_dims):
    f, v, d = tables.shape
    b = inputs.shape[0]
    a = weights.shape[-1]
    sample_dims = jnp.asarray(sample_dims, dtype=jnp.int32)

    tabT = jnp.transpose(tables, (0, 2, 1)).reshape(f * d, v)
    idxT = jnp.transpose(inputs, (1, 0))

    xt = _gather_cols(tabT, idxT)

    col = jnp.arange(d, dtype=jnp.int32)
    mask = (col[None, :] < sample_dims[:, None]).astype(jnp.float32)
    m2 = mask.reshape(f * d, 1)
    w2 = weights.reshape(f * d, a)
    ot = _matmul_t(xt, w2, m2, bm=2048)
    return jnp.transpose(ot, (1, 0))


# speedup vs baseline: 1.0397x; 1.0037x over previous
import functools

import jax
import jax.numpy as jnp
from jax import lax
from jax.experimental import pallas as pl
from jax.experimental.pallas import tpu as pltpu
from jax.experimental.pallas import tpu_sc as plsc

_NC = 2
_NS = 16
_NW = _NC * _NS

_OC = 4096


def _gather_cols(tabT, idxT):
    r_total, v = tabT.shape
    f_total, b_total = idxT.shape
    d = r_total // f_total
    rows_per_w = r_total // _NW
    n_chunks = b_total // _OC

    mesh = plsc.VectorSubcoreMesh(core_axis_name="c", subcore_axis_name="s")

    @functools.partial(
        pl.kernel,
        mesh=mesh,
        out_type=jax.ShapeDtypeStruct((r_total, b_total), jnp.float32),
        scratch_types=[
            pltpu.VMEM((v,), jnp.float32),
            pltpu.VMEM((b_total,), jnp.int32),
            pltpu.VMEM((_OC,), jnp.float32),
            pltpu.VMEM((_OC,), jnp.float32),
            pltpu.SemaphoreType.DMA,
            pltpu.SemaphoreType.DMA,
            pltpu.SemaphoreType.DMA,
        ],
        compiler_params=pltpu.CompilerParams(needs_layout_passes=False),
    )
    def k(tab_hbm, idx_hbm, out_hbm, row_v, idx_v, out0_v, out1_v, sem0, sem1,
          sem_row):
        wid = lax.axis_index("s") * _NC + lax.axis_index("c")
        out_bufs = (out0_v, out1_v)
        sems = (sem0, sem1)

        def row_body(rr, f_prev):
            r = wid * rows_per_w + rr
            f = r // d

            row_cp = pltpu.make_async_copy(tab_hbm.at[r], row_v, sem_row)
            row_cp.start()

            @pl.when(f != f_prev)
            def _():
                pltpu.sync_copy(idx_hbm.at[f], idx_v)

            @pl.when(rr > 0)
            def _():
                for h in range(n_chunks - 2, n_chunks):
                    pltpu.make_async_copy(
                        out_bufs[h % 2],
                        out_hbm.at[r, pl.ds(h * _OC, _OC)],
                        sems[h % 2],
                    ).wait()

            row_cp.wait()

            for h in range(n_chunks):
                buf = out_bufs[h % 2]
                sem = sems[h % 2]
                if h >= 2:
                    pltpu.make_async_copy(
                        buf, out_hbm.at[r, pl.ds((h - 2) * _OC, _OC)], sem
                    ).wait()

                @plsc.parallel_loop(0, _OC, 16, unroll=32)
                def gi(i):
                    idx16 = idx_v[pl.ds(h * _OC + i, 16)]
                    buf[pl.ds(i, 16)] = plsc.load_gather(row_v, [idx16])

                pltpu.make_async_copy(
                    buf, out_hbm.at[r, pl.ds(h * _OC, _OC)], sem
                ).start()

            return f

        lax.fori_loop(0, rows_per_w, row_body, jnp.int32(-1))
        for h in range(n_chunks - 2, n_chunks):
            pltpu.make_async_copy(
                out_bufs[h % 2],
                out_hbm.at[0, pl.ds(h * _OC, _OC)],
                sems[h % 2],
            ).wait()

    return k(tabT, idxT)


def _mm_body(xt_ref, w_ref, m_ref, o_ref):
    w = w_ref[...] * m_ref[...]
    o_ref[...] = lax.dot_general(
        w, xt_ref[...],
        dimension_numbers=(((0,), (0,)), ((), ())),
        preferred_element_type=jnp.float32,
    )


def _matmul_t(xt, w2, m2, bm):
    fd, b = xt.shape
    a = w2.shape[1]
    return pl.pallas_call(
        _mm_body,
        grid=(b // bm,),
        in_specs=[
            pl.BlockSpec((fd, bm), lambda i: (0, i)),
            pl.BlockSpec((fd, a), lambda i: (0, 0)),
            pl.BlockSpec((fd, 1), lambda i: (0, 0)),
        ],
        out_specs=pl.BlockSpec((a, bm), lambda i: (0, i)),
        out_shape=jax.ShapeDtypeStruct((a, b), jnp.float32),
    )(xt, w2, m2)


def kernel(inputs, tables, weights, sample_dims):
    f, v, d = tables.shape
    b = inputs.shape[0]
    a = weights.shape[-1]
    sample_dims = jnp.asarray(sample_dims, dtype=jnp.int32)

    tabT = jnp.transpose(tables, (0, 2, 1)).reshape(f * d, v)
    idxT = jnp.transpose(inputs, (1, 0))

    xt = _gather_cols(tabT, idxT)

    col = jnp.arange(d, dtype=jnp.int32)
    mask = (col[None, :] < sample_dims[:, None]).astype(jnp.float32)
    m2 = mask.reshape(f * d, 1)
    w2 = weights.reshape(f * d, a)
    ot = _matmul_t(xt, w2, m2, bm=2048)
    return jnp.transpose(ot, (1, 0))
